# baseline (speedup 1.0000x reference)
import functools

import jax
import jax.numpy as jnp
from jax import lax
from jax.experimental import pallas as pl
from jax.experimental.pallas import tpu as pltpu
from jax.experimental.pallas import tpu_sc as plsc

_B, _DIN, _DF, _C, _K = 16384, 512, 128, 1000, 100
_BLK = 512
_N = _C + _B
_NPAD = 17408
_RBLK = 1024
_PADCLS = _C


def _row_stats(p):
    m = jnp.max(p, axis=1, keepdims=True)
    ex = jnp.exp(p - m)
    se = jnp.sum(ex, axis=1, keepdims=True)
    ent = (m + jnp.log(se)) - jnp.sum(p * ex, axis=1, keepdims=True) / se
    ii = lax.broadcasted_iota(jnp.int32, p.shape, 1)
    cls = jnp.min(jnp.where(p == m, ii, jnp.int32(2**30)), axis=1,
                  keepdims=True)
    return cls, ent


def _feat_body(x_ref, wf_ref, bf_ref, wc_ref, bc_ref,
               z_ref, zn_ref, c_ref, e_ref):
    z = jnp.dot(x_ref[...], wf_ref[...],
                preferred_element_type=jnp.float32) + bf_ref[...]
    z_ref[...] = z
    nrm = jnp.sqrt(jnp.sum(z * z, axis=1, keepdims=True))
    zn_ref[...] = z / jnp.maximum(nrm, 1e-12)
    p = lax.dot_general(z, wc_ref[...], (((1,), (1,)), ((), ())),
                        preferred_element_type=jnp.float32) + bc_ref[...]
    cls, ent = _row_stats(p)
    c_ref[...] = cls
    e_ref[...] = ent


def _warm_body(wc_ref, bc_ref, wn_ref, c_ref, e_ref):
    w = wc_ref[...]
    nrm = jnp.sqrt(jnp.sum(w * w, axis=1, keepdims=True))
    wn_ref[...] = w / jnp.maximum(nrm, 1e-12)
    p = lax.dot_general(w, w, (((1,), (1,)), ((), ())),
                        preferred_element_type=jnp.float32) + bc_ref[...]
    cls, ent = _row_stats(p)
    c_ref[...] = cls
    e_ref[...] = ent


def _counts_body(c_ref, cnt_ref):
    @pl.when(pl.program_id(0) == 0)
    def _():
        cnt_ref[...] = jnp.zeros_like(cnt_ref)
    oh = (c_ref[...] == lax.broadcasted_iota(jnp.int32, (_RBLK, _C), 1))
    cnt_ref[...] += jnp.sum(oh.astype(jnp.int32), axis=0, keepdims=True)


def _rank_body(ci_ref, ei_ref, sup_ref, cr_ref, er_ref, out_ref):
    i = pl.program_id(0)
    ci = ci_ref[...]
    ei = ei_ref[...]
    i_idx = i * _RBLK + lax.broadcasted_iota(jnp.int32, (_RBLK, 1), 0)

    def body(j, cnt):
        cj = cr_ref[j]
        ej = er_ref[j]
        j_idx = j * _RBLK + lax.broadcasted_iota(jnp.int32, (1, _RBLK), 1)
        less = (ej < ei) | ((ej == ei) & (j_idx < i_idx))
        hit = (cj == ci) & less
        return cnt + jnp.sum(hit.astype(jnp.int32), axis=1, keepdims=True)

    cnt = lax.fori_loop(0, _NPAD // _RBLK, body,
                        jnp.zeros((_RBLK, 1), jnp.int32))
    out_ref[...] = sup_ref[...] * (cnt < _K).astype(jnp.float32)


_NACC = 1024
_PW = _NPAD // 32
_CH = (128, 128, 128, 128, 32)


def _sc_scatter_body(sup_hbm, c_hbm, zero_hbm, out_hbm,
                     rows_a, idx_a, rows_b, idx_b, shared):
    cid = lax.axis_index("c")
    sid = lax.axis_index("s")
    base = (sid * 2 + cid) * _PW

    @pl.when(sid == 0)
    def _():
        pltpu.sync_copy(zero_hbm, shared)

    plsc.subcore_barrier()
    off = 0
    for sz in _CH:
        rows, idx = (rows_a, idx_a) if sz == 128 else (rows_b, idx_b)
        pltpu.sync_copy(sup_hbm.at[pl.ds(base + off, sz), :], rows)
        pltpu.sync_copy(c_hbm.at[pl.ds(base + off, sz)], idx)
        pltpu.sync_copy(rows, shared.at[idx], add=True)
        off += sz
    plsc.subcore_barrier()
    nrow = _NACC // 16
    pltpu.sync_copy(shared.at[pl.ds(sid * nrow, nrow), :],
                    out_hbm.at[cid, pl.ds(sid * nrow, nrow), :])


def _out_body(z_ref, w_ref, o_ref):
    w = w_ref[0] + w_ref[1]
    nrm = jnp.sqrt(jnp.sum(w * w, axis=1, keepdims=True))
    wn = w / jnp.maximum(nrm, 1e-12)
    res = lax.dot_general(z_ref[...], wn, (((1,), (1,)), ((), ())),
                          preferred_element_type=jnp.float32)
    o_ref[...] = res[:, :_C]


def kernel(x, W_feat, b_feat, W_cls, b_cls):
    bf2 = b_feat.reshape(1, _DF)
    bc2 = b_cls.reshape(1, _C)

    z, z_n, c_b, e_b = pl.pallas_call(
        _feat_body,
        grid=(_B // _BLK,),
        in_specs=[
            pl.BlockSpec((_BLK, _DIN), lambda i: (i, 0)),
            pl.BlockSpec((_DIN, _DF), lambda i: (0, 0)),
            pl.BlockSpec((1, _DF), lambda i: (0, 0)),
            pl.BlockSpec((_C, _DF), lambda i: (0, 0)),
            pl.BlockSpec((1, _C), lambda i: (0, 0)),
        ],
        out_specs=[
            pl.BlockSpec((_BLK, _DF), lambda i: (i, 0)),
            pl.BlockSpec((_BLK, _DF), lambda i: (i, 0)),
            pl.BlockSpec((_BLK, 1), lambda i: (i, 0)),
            pl.BlockSpec((_BLK, 1), lambda i: (i, 0)),
        ],
        out_shape=[
            jax.ShapeDtypeStruct((_B, _DF), jnp.float32),
            jax.ShapeDtypeStruct((_B, _DF), jnp.float32),
            jax.ShapeDtypeStruct((_B, 1), jnp.int32),
            jax.ShapeDtypeStruct((_B, 1), jnp.float32),
        ],
    )(x, W_feat, bf2, W_cls, bc2)

    w_n, c_w, e_w = pl.pallas_call(
        _warm_body,
        out_shape=[
            jax.ShapeDtypeStruct((_C, _DF), jnp.float32),
            jax.ShapeDtypeStruct((_C, 1), jnp.int32),
            jax.ShapeDtypeStruct((_C, 1), jnp.float32),
        ],
    )(W_cls, bc2)

    npad = _NPAD - _N
    c_all = jnp.concatenate(
        [c_w, c_b, jnp.full((npad, 1), _PADCLS, jnp.int32)], axis=0)
    e_all = jnp.concatenate(
        [e_w, e_b, jnp.zeros((npad, 1), jnp.float32)], axis=0)
    sup_n = jnp.concatenate(
        [w_n, z_n, jnp.zeros((npad, _DF), jnp.float32)], axis=0)

    counts = pl.pallas_call(
        _counts_body,
        grid=(_NPAD // _RBLK,),
        in_specs=[pl.BlockSpec((_RBLK, 1), lambda i: (i, 0))],
        out_specs=pl.BlockSpec((1, _C), lambda i: (0, 0)),
        out_shape=jax.ShapeDtypeStruct((1, _C), jnp.int32),
    )(c_all)
    maxc = jnp.max(counts[0, :_C])

    def _rare(sup):
        cr = c_all.reshape(_NPAD // _RBLK, 1, _RBLK)
        er = e_all.reshape(_NPAD // _RBLK, 1, _RBLK)
        return pl.pallas_call(
            _rank_body,
            grid=(_NPAD // _RBLK,),
            in_specs=[
                pl.BlockSpec((_RBLK, 1), lambda i: (i, 0)),
                pl.BlockSpec((_RBLK, 1), lambda i: (i, 0)),
                pl.BlockSpec((_RBLK, _DF), lambda i: (i, 0)),
                pl.BlockSpec((_NPAD // _RBLK, 1, _RBLK),
                             lambda i: (0, 0, 0)),
                pl.BlockSpec((_NPAD // _RBLK, 1, _RBLK),
                             lambda i: (0, 0, 0)),
            ],
            out_specs=pl.BlockSpec((_RBLK, _DF), lambda i: (i, 0)),
            out_shape=jax.ShapeDtypeStruct((_NPAD, _DF), jnp.float32),
        )(c_all, e_all, sup, cr, er)

    sup_fin = lax.cond(maxc <= _K, lambda s: s, _rare, sup_n)

    sc_scatter = functools.partial(
        pl.kernel,
        mesh=plsc.VectorSubcoreMesh(core_axis_name="c", subcore_axis_name="s"),
        out_type=jax.ShapeDtypeStruct((2, _NACC, _DF), jnp.float32),
        scratch_types=[
            pltpu.VMEM((128, _DF), jnp.float32),
            pltpu.VMEM((128,), jnp.int32),
            pltpu.VMEM((32, _DF), jnp.float32),
            pltpu.VMEM((32,), jnp.int32),
            pltpu.VMEM_SHARED((_NACC, _DF), jnp.float32),
        ],
    )(_sc_scatter_body)
    w2 = sc_scatter(sup_fin, c_all.reshape(_NPAD),
                    jnp.zeros((_NACC, _DF), jnp.float32))

    out = pl.pallas_call(
        _out_body,
        grid=(_B // _BLK,),
        in_specs=[
            pl.BlockSpec((_BLK, _DF), lambda i: (i, 0)),
            pl.BlockSpec((2, _NACC, _DF), lambda i: (0, 0, 0)),
        ],
        out_specs=pl.BlockSpec((_BLK, _C), lambda i: (i, 0)),
        out_shape=jax.ShapeDtypeStruct((_B, _C), jnp.float32),
    )(z, w2)
    return out


# speedup vs baseline: 1.0565x; 1.0565x over previous
import functools

import jax
import jax.numpy as jnp
from jax import lax
from jax.experimental import pallas as pl
from jax.experimental.pallas import tpu as pltpu
from jax.experimental.pallas import tpu_sc as plsc

_B, _DIN, _DF, _C, _K = 16384, 512, 128, 1000, 100
_BLK = 512
_N = _C + _B
_NPAD = 17408
_RBLK = 1024
_PADCLS = _C


def _row_stats(p):
    m = jnp.max(p, axis=1, keepdims=True)
    ex = jnp.exp(p - m)
    se = jnp.sum(ex, axis=1, keepdims=True)
    ent = (m + jnp.log(se)) - jnp.sum(p * ex, axis=1, keepdims=True) / se
    ii = lax.broadcasted_iota(jnp.int32, p.shape, 1)
    cls = jnp.min(jnp.where(p == m, ii, jnp.int32(2**30)), axis=1,
                  keepdims=True)
    return cls, ent


def _counts_of(cls, nrows):
    oh = (cls == lax.broadcasted_iota(jnp.int32, (nrows, _C), 1))
    return jnp.sum(oh.astype(jnp.int32), axis=0, keepdims=True)


def _feat_body(x_ref, wf_ref, bf_ref, wc_ref, bc_ref,
               z_ref, zn_ref, c_ref, e_ref, cnt_ref):
    z = jnp.dot(x_ref[...], wf_ref[...],
                preferred_element_type=jnp.float32) + bf_ref[...]
    z_ref[...] = z
    nrm = jnp.sqrt(jnp.sum(z * z, axis=1, keepdims=True))
    zn_ref[...] = z / jnp.maximum(nrm, 1e-12)
    p = lax.dot_general(z, wc_ref[...], (((1,), (1,)), ((), ())),
                        preferred_element_type=jnp.float32) + bc_ref[...]
    cls, ent = _row_stats(p)
    c_ref[...] = cls
    e_ref[...] = ent
    cnt_ref[...] = _counts_of(cls, _BLK)[None]


def _warm_body(wc_ref, bc_ref, wn_ref, c_ref, e_ref, cnt_ref):
    w = wc_ref[...]
    nrm = jnp.sqrt(jnp.sum(w * w, axis=1, keepdims=True))
    wn_ref[...] = w / jnp.maximum(nrm, 1e-12)
    p = lax.dot_general(w, w, (((1,), (1,)), ((), ())),
                        preferred_element_type=jnp.float32) + bc_ref[...]
    cls, ent = _row_stats(p)
    c_ref[...] = cls
    e_ref[...] = ent
    cnt_ref[...] = _counts_of(cls, _C)


def _rank_body(ci_ref, ei_ref, sup_ref, cr_ref, er_ref, out_ref):
    i = pl.program_id(0)
    ci = ci_ref[...]
    ei = ei_ref[...]
    i_idx = i * _RBLK + lax.broadcasted_iota(jnp.int32, (_RBLK, 1), 0)

    def body(j, cnt):
        cj = cr_ref[j]
        ej = er_ref[j]
        j_idx = j * _RBLK + lax.broadcasted_iota(jnp.int32, (1, _RBLK), 1)
        less = (ej < ei) | ((ej == ei) & (j_idx < i_idx))
        hit = (cj == ci) & less
        return cnt + jnp.sum(hit.astype(jnp.int32), axis=1, keepdims=True)

    cnt = lax.fori_loop(0, _NPAD // _RBLK, body,
                        jnp.zeros((_RBLK, 1), jnp.int32))
    out_ref[...] = sup_ref[...] * (cnt < _K).astype(jnp.float32)


_NACC = 1024
_PW = _NPAD // 32
_CH = (128, 128, 128, 128, 32)


def _sc_scatter_body(sup_hbm, c_hbm, zero_hbm, out_hbm,
                     rows_a, idx_a, rows_b, idx_b, shared):
    cid = lax.axis_index("c")
    sid = lax.axis_index("s")
    base = (sid * 2 + cid) * _PW

    @pl.when(sid == 0)
    def _():
        pltpu.sync_copy(zero_hbm, shared)

    plsc.subcore_barrier()
    off = 0
    for sz in _CH:
        rows, idx = (rows_a, idx_a) if sz == 128 else (rows_b, idx_b)
        pltpu.sync_copy(sup_hbm.at[pl.ds(base + off, sz), :], rows)
        pltpu.sync_copy(c_hbm.at[pl.ds(base + off, sz)], idx)
        pltpu.sync_copy(rows, shared.at[idx], add=True)
        off += sz
    plsc.subcore_barrier()
    nrow = _NACC // 16
    pltpu.sync_copy(shared.at[pl.ds(sid * nrow, nrow), :],
                    out_hbm.at[cid, pl.ds(sid * nrow, nrow), :])


def _out_body(z_ref, w_ref, o_ref):
    w = w_ref[0] + w_ref[1]
    nrm = jnp.sqrt(jnp.sum(w * w, axis=1, keepdims=True))
    wn = w / jnp.maximum(nrm, 1e-12)
    res = lax.dot_general(z_ref[...], wn, (((1,), (1,)), ((), ())),
                          preferred_element_type=jnp.float32)
    o_ref[...] = res[:, :_C]


def kernel(x, W_feat, b_feat, W_cls, b_cls):
    bf2 = b_feat.reshape(1, _DF)
    bc2 = b_cls.reshape(1, _C)

    z, z_n, c_b, e_b, cnt_b = pl.pallas_call(
        _feat_body,
        grid=(_B // _BLK,),
        in_specs=[
            pl.BlockSpec((_BLK, _DIN), lambda i: (i, 0)),
            pl.BlockSpec((_DIN, _DF), lambda i: (0, 0)),
            pl.BlockSpec((1, _DF), lambda i: (0, 0)),
            pl.BlockSpec((_C, _DF), lambda i: (0, 0)),
            pl.BlockSpec((1, _C), lambda i: (0, 0)),
        ],
        out_specs=[
            pl.BlockSpec((_BLK, _DF), lambda i: (i, 0)),
            pl.BlockSpec((_BLK, _DF), lambda i: (i, 0)),
            pl.BlockSpec((_BLK, 1), lambda i: (i, 0)),
            pl.BlockSpec((_BLK, 1), lambda i: (i, 0)),
            pl.BlockSpec((1, 1, _C), lambda i: (i, 0, 0)),
        ],
        out_shape=[
            jax.ShapeDtypeStruct((_B, _DF), jnp.float32),
            jax.ShapeDtypeStruct((_B, _DF), jnp.float32),
            jax.ShapeDtypeStruct((_B, 1), jnp.int32),
            jax.ShapeDtypeStruct((_B, 1), jnp.float32),
            jax.ShapeDtypeStruct((_B // _BLK, 1, _C), jnp.int32),
        ],
        compiler_params=pltpu.CompilerParams(
            dimension_semantics=("parallel",)),
    )(x, W_feat, bf2, W_cls, bc2)

    w_n, c_w, e_w, cnt_w = pl.pallas_call(
        _warm_body,
        out_shape=[
            jax.ShapeDtypeStruct((_C, _DF), jnp.float32),
            jax.ShapeDtypeStruct((_C, 1), jnp.int32),
            jax.ShapeDtypeStruct((_C, 1), jnp.float32),
            jax.ShapeDtypeStruct((1, _C), jnp.int32),
        ],
    )(W_cls, bc2)

    npad = _NPAD - _N
    c_all = jnp.concatenate(
        [c_w, c_b, jnp.full((npad, 1), _PADCLS, jnp.int32)], axis=0)
    e_all = jnp.concatenate(
        [e_w, e_b, jnp.zeros((npad, 1), jnp.float32)], axis=0)
    sup_n = jnp.concatenate(
        [w_n, z_n, jnp.zeros((npad, _DF), jnp.float32)], axis=0)

    maxc = jnp.max(jnp.sum(cnt_b[:, 0, :], axis=0) + cnt_w[0])

    def _rare(sup):
        cr = c_all.reshape(_NPAD // _RBLK, 1, _RBLK)
        er = e_all.reshape(_NPAD // _RBLK, 1, _RBLK)
        return pl.pallas_call(
            _rank_body,
            grid=(_NPAD // _RBLK,),
            in_specs=[
                pl.BlockSpec((_RBLK, 1), lambda i: (i, 0)),
                pl.BlockSpec((_RBLK, 1), lambda i: (i, 0)),
                pl.BlockSpec((_RBLK, _DF), lambda i: (i, 0)),
                pl.BlockSpec((_NPAD // _RBLK, 1, _RBLK),
                             lambda i: (0, 0, 0)),
                pl.BlockSpec((_NPAD // _RBLK, 1, _RBLK),
                             lambda i: (0, 0, 0)),
            ],
            out_specs=pl.BlockSpec((_RBLK, _DF), lambda i: (i, 0)),
            out_shape=jax.ShapeDtypeStruct((_NPAD, _DF), jnp.float32),
        )(c_all, e_all, sup, cr, er)

    sup_fin = lax.cond(maxc <= _K, lambda s: s, _rare, sup_n)

    sc_scatter = functools.partial(
        pl.kernel,
        mesh=plsc.VectorSubcoreMesh(core_axis_name="c", subcore_axis_name="s"),
        out_type=jax.ShapeDtypeStruct((2, _NACC, _DF), jnp.float32),
        scratch_types=[
            pltpu.VMEM((128, _DF), jnp.float32),
            pltpu.VMEM((128,), jnp.int32),
            pltpu.VMEM((32, _DF), jnp.float32),
            pltpu.VMEM((32,), jnp.int32),
            pltpu.VMEM_SHARED((_NACC, _DF), jnp.float32),
        ],
    )(_sc_scatter_body)
    w2 = sc_scatter(sup_fin, c_all.reshape(_NPAD),
                    jnp.zeros((_NACC, _DF), jnp.float32))

    out = pl.pallas_call(
        _out_body,
        grid=(_B // _BLK,),
        in_specs=[
            pl.BlockSpec((_BLK, _DF), lambda i: (i, 0)),
            pl.BlockSpec((2, _NACC, _DF), lambda i: (0, 0, 0)),
        ],
        out_specs=pl.BlockSpec((_BLK, _C), lambda i: (i, 0)),
        out_shape=jax.ShapeDtypeStruct((_B, _C), jnp.float32),
        compiler_params=pltpu.CompilerParams(
            dimension_semantics=("parallel",)),
    )(z, w2)
    return out


# speedup vs baseline: 1.1550x; 1.0932x over previous
import functools

import jax
import jax.numpy as jnp
from jax import lax
from jax.experimental import pallas as pl
from jax.experimental.pallas import tpu as pltpu
from jax.experimental.pallas import tpu_sc as plsc

_B, _DIN, _DF, _C, _K = 16384, 512, 128, 1000, 100
_BLK = 512
_CPAD = 1024
_NPAD = _CPAD + _B
_RBLK = 1024
_PADCLS = _C


def _row_stats(p):
    m = jnp.max(p, axis=1, keepdims=True)
    ex = jnp.exp(p - m)
    se = jnp.sum(ex, axis=1, keepdims=True)
    ent = (m + jnp.log(se)) - jnp.sum(p * ex, axis=1, keepdims=True) / se
    ii = lax.broadcasted_iota(jnp.int32, p.shape, 1)
    cls = jnp.min(jnp.where(p == m, ii, jnp.int32(2**30)), axis=1,
                  keepdims=True)
    return cls, ent


def _counts_of(cls, nrows):
    oh = (cls == lax.broadcasted_iota(jnp.int32, (nrows, _C), 1))
    return jnp.sum(oh.astype(jnp.int32), axis=0, keepdims=True)


def _feat_body(x_ref, wf_ref, bf_ref, wc_ref, bc_ref,
               z_ref, zn_ref, c_ref, e_ref, cnt_ref):
    z = jnp.dot(x_ref[...], wf_ref[...],
                preferred_element_type=jnp.float32) + bf_ref[...]
    z_ref[...] = z
    nrm = jnp.sqrt(jnp.sum(z * z, axis=1, keepdims=True))
    zn_ref[...] = z / jnp.maximum(nrm, 1e-12)
    p = lax.dot_general(z, wc_ref[...], (((1,), (1,)), ((), ())),
                        preferred_element_type=jnp.float32) + bc_ref[...]
    cls, ent = _row_stats(p)
    c_ref[...] = cls
    e_ref[...] = ent
    cnt_ref[...] = _counts_of(cls, _BLK)[None]


def _warm_body(wp_ref, wc_ref, bc_ref, wn_ref, c_ref, e_ref, cnt_ref):
    w = wp_ref[...]
    nrm = jnp.sqrt(jnp.sum(w * w, axis=1, keepdims=True))
    wn_ref[...] = w / jnp.maximum(nrm, 1e-12)
    p = lax.dot_general(w, wc_ref[...], (((1,), (1,)), ((), ())),
                        preferred_element_type=jnp.float32) + bc_ref[...]
    cls, ent = _row_stats(p)
    ridx = lax.broadcasted_iota(jnp.int32, (_CPAD, 1), 0)
    c_ref[...] = jnp.where(ridx < _C, cls, _PADCLS)
    e_ref[...] = ent
    cnt_ref[...] = _counts_of(c_ref[...], _CPAD)


def _rank_body(ci_ref, ei_ref, sup_ref, cr_ref, er_ref, out_ref):
    i = pl.program_id(0)
    ci = ci_ref[...]
    ei = ei_ref[...]
    i_idx = i * _RBLK + lax.broadcasted_iota(jnp.int32, (_RBLK, 1), 0)

    def body(j, cnt):
        cj = cr_ref[j]
        ej = er_ref[j]
        j_idx = j * _RBLK + lax.broadcasted_iota(jnp.int32, (1, _RBLK), 1)
        less = (ej < ei) | ((ej == ei) & (j_idx < i_idx))
        hit = (cj == ci) & less
        return cnt + jnp.sum(hit.astype(jnp.int32), axis=1, keepdims=True)

    cnt = lax.fori_loop(0, _NPAD // _RBLK, body,
                        jnp.zeros((_RBLK, 1), jnp.int32))
    out_ref[...] = sup_ref[...] * (cnt < _K).astype(jnp.float32)


_NACC = 1024


def _sc_scatter_body(wn_hbm, zn_hbm, cw_hbm, cb_hbm, zero_hbm, out_hbm,
                     rows_a, idx_a, rows_b, idx_b, shared):
    cid = lax.axis_index("c")
    sid = lax.axis_index("s")
    wid = sid * 2 + cid
    base_w = wid * (_CPAD // 32)
    base_z = wid * (_B // 32)

    @pl.when(sid == 0)
    def _():
        pltpu.sync_copy(zero_hbm, shared)

    plsc.subcore_barrier()
    pltpu.sync_copy(wn_hbm.at[pl.ds(base_w, _CPAD // 32), :], rows_b)
    pltpu.sync_copy(cw_hbm.at[pl.ds(base_w, _CPAD // 32)], idx_b)
    pltpu.sync_copy(rows_b, shared.at[idx_b], add=True)
    for k in range(_B // 32 // 128):
        pltpu.sync_copy(zn_hbm.at[pl.ds(base_z + k * 128, 128), :], rows_a)
        pltpu.sync_copy(cb_hbm.at[pl.ds(base_z + k * 128, 128)], idx_a)
        pltpu.sync_copy(rows_a, shared.at[idx_a], add=True)
    plsc.subcore_barrier()
    nrow = _NACC // 16
    pltpu.sync_copy(shared.at[pl.ds(sid * nrow, nrow), :],
                    out_hbm.at[cid, pl.ds(sid * nrow, nrow), :])


def _out_body(z_ref, w_ref, o_ref):
    w = w_ref[0] + w_ref[1]
    nrm = jnp.sqrt(jnp.sum(w * w, axis=1, keepdims=True))
    wn = w / jnp.maximum(nrm, 1e-12)
    res = lax.dot_general(z_ref[...], wn, (((1,), (1,)), ((), ())),
                          preferred_element_type=jnp.float32)
    o_ref[...] = res[:, :_C]


def kernel(x, W_feat, b_feat, W_cls, b_cls):
    bf2 = b_feat.reshape(1, _DF)
    bc2 = b_cls.reshape(1, _C)

    z, z_n, c_b, e_b, cnt_b = pl.pallas_call(
        _feat_body,
        grid=(_B // _BLK,),
        in_specs=[
            pl.BlockSpec((_BLK, _DIN), lambda i: (i, 0)),
            pl.BlockSpec((_DIN, _DF), lambda i: (0, 0)),
            pl.BlockSpec((1, _DF), lambda i: (0, 0)),
            pl.BlockSpec((_C, _DF), lambda i: (0, 0)),
            pl.BlockSpec((1, _C), lambda i: (0, 0)),
        ],
        out_specs=[
            pl.BlockSpec((_BLK, _DF), lambda i: (i, 0)),
            pl.BlockSpec((_BLK, _DF), lambda i: (i, 0)),
            pl.BlockSpec((_BLK, 1), lambda i: (i, 0)),
            pl.BlockSpec((_BLK, 1), lambda i: (i, 0)),
            pl.BlockSpec((1, 1, _C), lambda i: (i, 0, 0)),
        ],
        out_shape=[
            jax.ShapeDtypeStruct((_B, _DF), jnp.float32),
            jax.ShapeDtypeStruct((_B, _DF), jnp.float32),
            jax.ShapeDtypeStruct((_B, 1), jnp.int32),
            jax.ShapeDtypeStruct((_B, 1), jnp.float32),
            jax.ShapeDtypeStruct((_B // _BLK, 1, _C), jnp.int32),
        ],
        compiler_params=pltpu.CompilerParams(
            dimension_semantics=("parallel",)),
    )(x, W_feat, bf2, W_cls, bc2)

    Wp = jnp.concatenate(
        [W_cls, jnp.zeros((_CPAD - _C, _DF), jnp.float32)], axis=0)
    w_n, c_w, e_w, cnt_w = pl.pallas_call(
        _warm_body,
        out_shape=[
            jax.ShapeDtypeStruct((_CPAD, _DF), jnp.float32),
            jax.ShapeDtypeStruct((_CPAD, 1), jnp.int32),
            jax.ShapeDtypeStruct((_CPAD, 1), jnp.float32),
            jax.ShapeDtypeStruct((1, _C), jnp.int32),
        ],
    )(Wp, W_cls, bc2)

    maxc = jnp.max(jnp.sum(cnt_b[:, 0, :], axis=0) + cnt_w[0])

    def _rare(ops):
        wn, zn = ops
        c_all = jnp.concatenate([c_w, c_b], axis=0)
        e_all = jnp.concatenate([e_w, e_b], axis=0)
        sup = jnp.concatenate([wn, zn], axis=0)
        cr = c_all.reshape(_NPAD // _RBLK, 1, _RBLK)
        er = e_all.reshape(_NPAD // _RBLK, 1, _RBLK)
        fin = pl.pallas_call(
            _rank_body,
            grid=(_NPAD // _RBLK,),
            in_specs=[
                pl.BlockSpec((_RBLK, 1), lambda i: (i, 0)),
                pl.BlockSpec((_RBLK, 1), lambda i: (i, 0)),
                pl.BlockSpec((_RBLK, _DF), lambda i: (i, 0)),
                pl.BlockSpec((_NPAD // _RBLK, 1, _RBLK),
                             lambda i: (0, 0, 0)),
                pl.BlockSpec((_NPAD // _RBLK, 1, _RBLK),
                             lambda i: (0, 0, 0)),
            ],
            out_specs=pl.BlockSpec((_RBLK, _DF), lambda i: (i, 0)),
            out_shape=jax.ShapeDtypeStruct((_NPAD, _DF), jnp.float32),
        )(c_all, e_all, sup, cr, er)
        return fin[:_CPAD], fin[_CPAD:]

    wn_fin, zn_fin = lax.cond(maxc <= _K, lambda ops: ops, _rare, (w_n, z_n))

    sc_scatter = functools.partial(
        pl.kernel,
        mesh=plsc.VectorSubcoreMesh(core_axis_name="c", subcore_axis_name="s"),
        out_type=jax.ShapeDtypeStruct((2, _NACC, _DF), jnp.float32),
        scratch_types=[
            pltpu.VMEM((128, _DF), jnp.float32),
            pltpu.VMEM((128,), jnp.int32),
            pltpu.VMEM((_CPAD // 32, _DF), jnp.float32),
            pltpu.VMEM((_CPAD // 32,), jnp.int32),
            pltpu.VMEM_SHARED((_NACC, _DF), jnp.float32),
        ],
    )(_sc_scatter_body)
    w2 = sc_scatter(wn_fin, zn_fin, c_w.reshape(_CPAD), c_b.reshape(_B),
                    jnp.zeros((_NACC, _DF), jnp.float32))

    out = pl.pallas_call(
        _out_body,
        grid=(_B // _BLK,),
        in_specs=[
            pl.BlockSpec((_BLK, _DF), lambda i: (i, 0)),
            pl.BlockSpec((2, _NACC, _DF), lambda i: (0, 0, 0)),
        ],
        out_specs=pl.BlockSpec((_BLK, _C), lambda i: (i, 0)),
        out_shape=jax.ShapeDtypeStruct((_B, _C), jnp.float32),
        compiler_params=pltpu.CompilerParams(
            dimension_semantics=("parallel",)),
    )(z, w2)
    return out


# speedup vs baseline: 1.2418x; 1.0751x over previous
import functools

import jax
import jax.numpy as jnp
from jax import lax
from jax.experimental import pallas as pl
from jax.experimental.pallas import tpu as pltpu
from jax.experimental.pallas import tpu_sc as plsc

_B, _DIN, _DF, _C, _K = 16384, 512, 128, 1000, 100
_BLK = 1024
_CPAD = 1024
_NPAD = _CPAD + _B
_RBLK = 1024
_PADCLS = _C


def _row_stats(p):
    m = jnp.max(p, axis=1, keepdims=True)
    ex = jnp.exp(p - m)
    se = jnp.sum(ex, axis=1, keepdims=True)
    ent = (m + jnp.log(se)) - jnp.sum(p * ex, axis=1, keepdims=True) / se
    ii = lax.broadcasted_iota(jnp.int32, p.shape, 1)
    cls = jnp.min(jnp.where(p == m, ii, jnp.int32(2**30)), axis=1,
                  keepdims=True)
    return cls, ent


def _counts_of(cls, nrows):
    oh = (cls == lax.broadcasted_iota(jnp.int32, (nrows, _C), 1))
    return jnp.sum(oh.astype(jnp.int32), axis=0, keepdims=True)


def _feat_body(x_ref, wf_ref, bf_ref, wc_ref, bc_ref,
               z_ref, zn_ref, c_ref, e_ref, cnt_ref):
    z = jnp.dot(x_ref[...], wf_ref[...],
                preferred_element_type=jnp.float32) + bf_ref[...]
    z_ref[...] = z
    nrm = jnp.sqrt(jnp.sum(z * z, axis=1, keepdims=True))
    zn_ref[...] = z / jnp.maximum(nrm, 1e-12)
    p = lax.dot_general(z, wc_ref[...], (((1,), (1,)), ((), ())),
                        preferred_element_type=jnp.float32) + bc_ref[...]
    cls, ent = _row_stats(p)
    c_ref[...] = cls
    e_ref[...] = ent
    cnt_ref[...] = _counts_of(cls, _BLK)[None]


def _warm_body(wp_ref, wc_ref, bc_ref, wn_ref, c_ref, e_ref, cnt_ref):
    w = wp_ref[...]
    nrm = jnp.sqrt(jnp.sum(w * w, axis=1, keepdims=True))
    wn_ref[...] = w / jnp.maximum(nrm, 1e-12)
    p = lax.dot_general(w, wc_ref[...], (((1,), (1,)), ((), ())),
                        preferred_element_type=jnp.float32) + bc_ref[...]
    cls, ent = _row_stats(p)
    ridx = lax.broadcasted_iota(jnp.int32, (_CPAD, 1), 0)
    c_ref[...] = jnp.where(ridx < _C, cls, _PADCLS)
    e_ref[...] = ent
    cnt_ref[...] = _counts_of(c_ref[...], _CPAD)


def _rank_body(ci_ref, ei_ref, sup_ref, cr_ref, er_ref, out_ref):
    i = pl.program_id(0)
    ci = ci_ref[...]
    ei = ei_ref[...]
    i_idx = i * _RBLK + lax.broadcasted_iota(jnp.int32, (_RBLK, 1), 0)

    def body(j, cnt):
        cj = cr_ref[j]
        ej = er_ref[j]
        j_idx = j * _RBLK + lax.broadcasted_iota(jnp.int32, (1, _RBLK), 1)
        less = (ej < ei) | ((ej == ei) & (j_idx < i_idx))
        hit = (cj == ci) & less
        return cnt + jnp.sum(hit.astype(jnp.int32), axis=1, keepdims=True)

    cnt = lax.fori_loop(0, _NPAD // _RBLK, body,
                        jnp.zeros((_RBLK, 1), jnp.int32))
    out_ref[...] = sup_ref[...] * (cnt < _K).astype(jnp.float32)


_NACC = 1024


def _sc_scatter_body(wn_hbm, zn_hbm, cw_hbm, cb_hbm, zero_hbm, out_hbm,
                     rows_a, idx_a, rows_b, idx_b, shared):
    cid = lax.axis_index("c")
    sid = lax.axis_index("s")
    wid = sid * 2 + cid
    base_w = wid * (_CPAD // 32)
    base_z = wid * (_B // 32)

    @pl.when(sid == 0)
    def _():
        pltpu.sync_copy(zero_hbm, shared)

    plsc.subcore_barrier()
    pltpu.sync_copy(wn_hbm.at[pl.ds(base_w, _CPAD // 32), :], rows_b)
    pltpu.sync_copy(cw_hbm.at[pl.ds(base_w, _CPAD // 32)], idx_b)
    pltpu.sync_copy(rows_b, shared.at[idx_b], add=True)
    for k in range(_B // 32 // 128):
        pltpu.sync_copy(zn_hbm.at[pl.ds(base_z + k * 128, 128), :], rows_a)
        pltpu.sync_copy(cb_hbm.at[pl.ds(base_z + k * 128, 128)], idx_a)
        pltpu.sync_copy(rows_a, shared.at[idx_a], add=True)
    plsc.subcore_barrier()
    nrow = _NACC // 16
    pltpu.sync_copy(shared.at[pl.ds(sid * nrow, nrow), :],
                    out_hbm.at[cid, pl.ds(sid * nrow, nrow), :])


def _out_body(z_ref, w_ref, o_ref):
    w = w_ref[0] + w_ref[1]
    nrm = jnp.sqrt(jnp.sum(w * w, axis=1, keepdims=True))
    wn = w / jnp.maximum(nrm, 1e-12)
    res = lax.dot_general(z_ref[...], wn, (((1,), (1,)), ((), ())),
                          preferred_element_type=jnp.float32)
    o_ref[...] = res[:, :_C]


def kernel(x, W_feat, b_feat, W_cls, b_cls):
    bf2 = b_feat.reshape(1, _DF)
    bc2 = b_cls.reshape(1, _C)

    z, z_n, c_b, e_b, cnt_b = pl.pallas_call(
        _feat_body,
        grid=(_B // _BLK,),
        in_specs=[
            pl.BlockSpec((_BLK, _DIN), lambda i: (i, 0)),
            pl.BlockSpec((_DIN, _DF), lambda i: (0, 0)),
            pl.BlockSpec((1, _DF), lambda i: (0, 0)),
            pl.BlockSpec((_C, _DF), lambda i: (0, 0)),
            pl.BlockSpec((1, _C), lambda i: (0, 0)),
        ],
        out_specs=[
            pl.BlockSpec((_BLK, _DF), lambda i: (i, 0)),
            pl.BlockSpec((_BLK, _DF), lambda i: (i, 0)),
            pl.BlockSpec((_BLK, 1), lambda i: (i, 0)),
            pl.BlockSpec((_BLK, 1), lambda i: (i, 0)),
            pl.BlockSpec((1, 1, _C), lambda i: (i, 0, 0)),
        ],
        out_shape=[
            jax.ShapeDtypeStruct((_B, _DF), jnp.float32),
            jax.ShapeDtypeStruct((_B, _DF), jnp.float32),
            jax.ShapeDtypeStruct((_B, 1), jnp.int32),
            jax.ShapeDtypeStruct((_B, 1), jnp.float32),
            jax.ShapeDtypeStruct((_B // _BLK, 1, _C), jnp.int32),
        ],
        compiler_params=pltpu.CompilerParams(
            dimension_semantics=("parallel",)),
    )(x, W_feat, bf2, W_cls, bc2)

    Wp = jnp.concatenate(
        [W_cls, jnp.zeros((_CPAD - _C, _DF), jnp.float32)], axis=0)
    w_n, c_w, e_w, cnt_w = pl.pallas_call(
        _warm_body,
        out_shape=[
            jax.ShapeDtypeStruct((_CPAD, _DF), jnp.float32),
            jax.ShapeDtypeStruct((_CPAD, 1), jnp.int32),
            jax.ShapeDtypeStruct((_CPAD, 1), jnp.float32),
            jax.ShapeDtypeStruct((1, _C), jnp.int32),
        ],
    )(Wp, W_cls, bc2)

    maxc = jnp.max(jnp.sum(cnt_b[:, 0, :], axis=0) + cnt_w[0])

    def _rare(ops):
        wn, zn = ops
        c_all = jnp.concatenate([c_w, c_b], axis=0)
        e_all = jnp.concatenate([e_w, e_b], axis=0)
        sup = jnp.concatenate([wn, zn], axis=0)
        cr = c_all.reshape(_NPAD // _RBLK, 1, _RBLK)
        er = e_all.reshape(_NPAD // _RBLK, 1, _RBLK)
        fin = pl.pallas_call(
            _rank_body,
            grid=(_NPAD // _RBLK,),
            in_specs=[
                pl.BlockSpec((_RBLK, 1), lambda i: (i, 0)),
                pl.BlockSpec((_RBLK, 1), lambda i: (i, 0)),
                pl.BlockSpec((_RBLK, _DF), lambda i: (i, 0)),
                pl.BlockSpec((_NPAD // _RBLK, 1, _RBLK),
                             lambda i: (0, 0, 0)),
                pl.BlockSpec((_NPAD // _RBLK, 1, _RBLK),
                             lambda i: (0, 0, 0)),
            ],
            out_specs=pl.BlockSpec((_RBLK, _DF), lambda i: (i, 0)),
            out_shape=jax.ShapeDtypeStruct((_NPAD, _DF), jnp.float32),
        )(c_all, e_all, sup, cr, er)
        return fin[:_CPAD], fin[_CPAD:]

    wn_fin, zn_fin = lax.cond(maxc <= _K, lambda ops: ops, _rare, (w_n, z_n))

    sc_scatter = functools.partial(
        pl.kernel,
        mesh=plsc.VectorSubcoreMesh(core_axis_name="c", subcore_axis_name="s"),
        out_type=jax.ShapeDtypeStruct((2, _NACC, _DF), jnp.float32),
        scratch_types=[
            pltpu.VMEM((128, _DF), jnp.float32),
            pltpu.VMEM((128,), jnp.int32),
            pltpu.VMEM((_CPAD // 32, _DF), jnp.float32),
            pltpu.VMEM((_CPAD // 32,), jnp.int32),
            pltpu.VMEM_SHARED((_NACC, _DF), jnp.float32),
        ],
    )(_sc_scatter_body)
    w2 = sc_scatter(wn_fin, zn_fin, c_w.reshape(_CPAD), c_b.reshape(_B),
                    jnp.zeros((_NACC, _DF), jnp.float32))

    out = pl.pallas_call(
        _out_body,
        grid=(_B // _BLK,),
        in_specs=[
            pl.BlockSpec((_BLK, _DF), lambda i: (i, 0)),
            pl.BlockSpec((2, _NACC, _DF), lambda i: (0, 0, 0)),
        ],
        out_specs=pl.BlockSpec((_BLK, _C), lambda i: (i, 0)),
        out_shape=jax.ShapeDtypeStruct((_B, _C), jnp.float32),
        compiler_params=pltpu.CompilerParams(
            dimension_semantics=("parallel",)),
    )(z, w2)
    return out
